# Initial kernel scaffold; baseline (speedup 1.0000x reference)
import jax
import jax.numpy as jnp
from jax.experimental import pallas as pl


def kernel(student, teacher, edge_index):
    raise NotImplementedError("write your pallas kernel here")


# speedup vs baseline: 1.6747x; 1.6747x over previous
import functools

import jax
import jax.numpy as jnp
from jax import lax
from jax.experimental import pallas as pl
from jax.experimental.pallas import tpu as pltpu

N = 4096
D = 64
TOPK = 8
NCENT = 64
NSEED = 3
NITER = 10

ROWS_BLK = 256


def _l2n(x):
    return x / jnp.clip(jnp.linalg.norm(x, axis=-1, keepdims=True), 1e-12, None)


def _kmeans_labels(x, k, niter, seed):
    key = jax.random.key(seed)
    init_idx = jax.random.choice(key, x.shape[0], shape=(k,), replace=False)
    cent = x[init_idx]
    for _ in range(niter):
        d2 = jnp.sum(cent ** 2, axis=1)[None, :] - 2.0 * (x @ cent.T)
        labels = jnp.argmin(d2, axis=1)
        sums = jax.ops.segment_sum(x, labels, num_segments=k)
        counts = jax.ops.segment_sum(jnp.ones((x.shape[0],), x.dtype), labels, num_segments=k)
        cent = sums / jnp.maximum(counts, 1.0)[:, None]
    d2 = jnp.sum(cent ** 2, axis=1)[None, :] - 2.0 * (x @ cent.T)
    return jnp.argmin(d2, axis=1)


def _topk_body(s_ref, t_ref, vals_ref, idx_ref):
    i = pl.program_id(0)
    sim = lax.dot_general(
        s_ref[...], t_ref[...], (((1,), (1,)), ((), ())),
        preferred_element_type=jnp.float32)
    rows = lax.broadcasted_iota(jnp.int32, (ROWS_BLK, N), 0) + i * ROWS_BLK
    cols = lax.broadcasted_iota(jnp.int32, (ROWS_BLK, N), 1)
    sim = sim + jnp.where(rows == cols, jnp.float32(10.0), jnp.float32(0.0))
    cur = sim
    vs, ids = [], []
    for _ in range(TOPK):
        m = jnp.max(cur, axis=1, keepdims=True)
        amin = jnp.min(jnp.where(cur == m, cols, N), axis=1, keepdims=True)
        vs.append(m)
        ids.append(amin)
        cur = jnp.where(cols == amin, -jnp.inf, cur)
    vals_ref[...] = jnp.concatenate(vs, axis=1)
    idx_ref[...] = jnp.concatenate(ids, axis=1)


def _topk_tc(s, t):
    grid = (N // ROWS_BLK,)
    return pl.pallas_call(
        _topk_body,
        grid=grid,
        in_specs=[
            pl.BlockSpec((ROWS_BLK, D), lambda i: (i, 0)),
            pl.BlockSpec((N, D), lambda i: (0, 0)),
        ],
        out_specs=[
            pl.BlockSpec((ROWS_BLK, TOPK), lambda i: (i, 0)),
            pl.BlockSpec((ROWS_BLK, TOPK), lambda i: (i, 0)),
        ],
        out_shape=[
            jax.ShapeDtypeStruct((N, TOPK), jnp.float32),
            jax.ShapeDtypeStruct((N, TOPK), jnp.int32),
        ],
    )(s, t)


def kernel(student, teacher, edge_index):
    n, d = student.shape
    s = _l2n(student)
    t = _l2n(teacher)
    vals, I_knn = _topk_tc(s, t)
    t_sg = lax.stop_gradient(t)
    labels = jnp.stack(
        [_kmeans_labels(t_sg, NCENT, NITER, 1234 + si) for si in range(NSEED)], axis=0)
    batch_labels = labels[:, :, None]
    top_labels = labels[:, I_knn]
    close = jnp.any(batch_labels == top_labels, axis=0)
    rows = jnp.arange(n)[:, None]
    knn_dense = jnp.zeros((n, n), jnp.float32).at[rows, I_knn].add(1.0)
    adj = jnp.zeros((n, n), jnp.float32).at[edge_index[0], edge_index[1]].set(1.0)
    locality = knn_dense * adj
    globality = jnp.zeros((n, n), jnp.float32).at[rows, I_knn].add(close.astype(jnp.float32))
    pos = locality + globality
    return (vals, I_knn, pos)
